# Initial kernel scaffold; baseline (speedup 1.0000x reference)
import functools

import jax
import jax.numpy as jnp
from jax import lax
from jax.experimental import pallas as pl
from jax.experimental.pallas import tpu as pltpu
from jax.experimental.pallas import tpu_sc as plsc

N = 10000
E = 320000
D = 128

NUM_CORES = 2
NUM_SUBCORES = 16
NW = NUM_CORES * NUM_SUBCORES
CHUNK = 128
K = 79
EW = K * CHUNK
PE = NW * EW
NP = 10240

ROWS_PER_TILE = NP // NUM_SUBCORES


def _sc_segment_sum(with_deg: bool):
    mesh = plsc.VectorSubcoreMesh(core_axis_name="c", subcore_axis_name="s")
    out_type = [jax.ShapeDtypeStruct((NUM_CORES, NP, D), jnp.float32)]
    if with_deg:
        out_type.append(jax.ShapeDtypeStruct((NUM_CORES, NP), jnp.float32))

    scratch = [
        pltpu.VMEM((K, CHUNK), jnp.int32),
        pltpu.VMEM((K, CHUNK), jnp.int32),
        pltpu.VMEM((CHUNK, D), jnp.float32),
        pltpu.VMEM((CHUNK,), jnp.float32),
        pltpu.VMEM((ROWS_PER_TILE,), jnp.float32),
        pltpu.VMEM_SHARED((NP, D), jnp.float32),
        pltpu.VMEM_SHARED((NP,), jnp.float32),
        pltpu.SemaphoreType.DMA,
    ]

    @functools.partial(pl.kernel, mesh=mesh, out_type=out_type,
                       scratch_types=scratch)
    def body(x_hbm, src_hbm, dst_hbm, sums_out, *rest):
        if with_deg:
            deg_out = rest[0]
            (src_v, dst_v, rows_v, ones_v, zdeg_v, acc_sh, deg_sh, sem) = rest[1:]
        else:
            (src_v, dst_v, rows_v, ones_v, zdeg_v, acc_sh, deg_sh, sem) = rest

        cid = lax.axis_index("c")
        sid = lax.axis_index("s")
        wid = cid * NUM_SUBCORES + sid

        zeros16 = jnp.zeros((16,), jnp.float32)

        def zero_row(i, _):
            for j in range(D // 16):
                rows_v[i, pl.ds(j * 16, 16)] = zeros16
            return 0

        lax.fori_loop(0, CHUNK, zero_row, 0)

        def zero_deg(i, _):
            zdeg_v[pl.ds(i * 16, 16)] = zeros16
            return 0

        lax.fori_loop(0, ROWS_PER_TILE // 16, zero_deg, 0)

        def fill_ones(i, _):
            ones_v[pl.ds(i * 16, 16)] = zeros16 + 1.0
            return 0

        lax.fori_loop(0, CHUNK // 16, fill_ones, 0)

        row0 = sid * ROWS_PER_TILE
        for t in range(ROWS_PER_TILE // CHUNK):
            pltpu.sync_copy(rows_v, acc_sh.at[pl.ds(row0 + t * CHUNK, CHUNK)])
        pltpu.sync_copy(zdeg_v, deg_sh.at[pl.ds(row0, ROWS_PER_TILE)])

        pltpu.sync_copy(src_hbm.at[pl.ds(wid * K, K)], src_v)
        pltpu.sync_copy(dst_hbm.at[pl.ds(wid * K, K)], dst_v)

        plsc.subcore_barrier()

        def step(j, _):
            pltpu.async_copy(x_hbm.at[src_v.at[j]], rows_v, sem).wait()
            pltpu.sync_copy(rows_v, acc_sh.at[dst_v.at[j]], add=True)
            if with_deg:
                pltpu.sync_copy(ones_v, deg_sh.at[dst_v.at[j]], add=True)
            return 0

        lax.fori_loop(0, K, step, 0)

        plsc.subcore_barrier()

        pltpu.sync_copy(acc_sh.at[pl.ds(row0, ROWS_PER_TILE)],
                        sums_out.at[cid, pl.ds(row0, ROWS_PER_TILE)])
        if with_deg:
            pltpu.sync_copy(deg_sh.at[pl.ds(row0, ROWS_PER_TILE)],
                            deg_out.at[cid, pl.ds(row0, ROWS_PER_TILE)])

    return body


_sc_sum_deg = _sc_segment_sum(True)
_sc_sum = _sc_segment_sum(False)

BN = 1280


def _tc_layer_body(do_relu, sums_ref, deg_ref, x_ref, wl_ref, b_ref, wr_ref,
                   out_ref):
    s = sums_ref[0] + sums_ref[1]
    d = deg_ref[0] + deg_ref[1]
    inv = 1.0 / jnp.maximum(d, 1.0)
    agg = s * inv[:, None]
    r = (jnp.dot(agg, wl_ref[...], preferred_element_type=jnp.float32)
         + b_ref[...]
         + jnp.dot(x_ref[...], wr_ref[...], preferred_element_type=jnp.float32))
    if do_relu:
        r = jnp.maximum(r, 0.0)
    out_ref[...] = r


def _tc_layer(sums, deg, x, wl_t, b, wr_t, do_relu):
    grid = (NP // BN,)
    return pl.pallas_call(
        functools.partial(_tc_layer_body, do_relu),
        grid=grid,
        in_specs=[
            pl.BlockSpec((NUM_CORES, BN, D), lambda i: (0, i, 0)),
            pl.BlockSpec((NUM_CORES, BN), lambda i: (0, i)),
            pl.BlockSpec((BN, D), lambda i: (i, 0)),
            pl.BlockSpec((D, D), lambda i: (0, 0)),
            pl.BlockSpec((1, D), lambda i: (0, 0)),
            pl.BlockSpec((D, D), lambda i: (0, 0)),
        ],
        out_specs=pl.BlockSpec((BN, D), lambda i: (i, 0)),
        out_shape=jax.ShapeDtypeStruct((NP, D), jnp.float32),
    )(sums, deg, x, wl_t, b, wr_t)


def kernel(x, edge_index, W_l1, b_l1, W_r1, W_l2, b_l2, W_r2):
    src = edge_index[0]
    dst = edge_index[1]

    pad_n = PE - E
    pad_ar = jnp.arange(pad_n, dtype=jnp.int32)
    src_p = jnp.concatenate([src, pad_ar % N]).reshape(NW * K, CHUNK)
    dst_p = jnp.concatenate([dst, N + pad_ar % (NP - N)]).reshape(NW * K, CHUNK)

    x_p = jnp.concatenate([x, jnp.zeros((NP - N, D), jnp.float32)], axis=0)

    sums1, deg = _sc_sum_deg(x_p, src_p, dst_p)
    h = _tc_layer(sums1, deg, x_p, W_l1.T, b_l1.reshape(1, D), W_r1.T, True)
    (sums2,) = _sc_sum(h, src_p, dst_p)
    out = _tc_layer(sums2, deg, h, W_l2.T, b_l2.reshape(1, D), W_r2.T, False)
    return out[:N]


# speedup vs baseline: 9.1994x; 9.1994x over previous
import functools

import jax
import jax.numpy as jnp
from jax import lax
from jax.experimental import pallas as pl
from jax.experimental.pallas import tpu as pltpu
from jax.experimental.pallas import tpu_sc as plsc

N = 10000
E = 320000
D = 128

NUM_CORES = 2
NUM_SUBCORES = 16
NW = NUM_CORES * NUM_SUBCORES
CHUNK = 128
K = 80
EW = K * CHUNK
PE = NW * EW
NP = 10240

ROWS_PER_TILE = NP // NUM_SUBCORES


def _sc_segment_sum(with_deg: bool):
    mesh = plsc.VectorSubcoreMesh(core_axis_name="c", subcore_axis_name="s")
    out_type = [jax.ShapeDtypeStruct((NUM_CORES, NP, D), jnp.float32)]
    if with_deg:
        out_type.append(jax.ShapeDtypeStruct((NUM_CORES, NP), jnp.float32))

    scratch = [
        pltpu.VMEM((K, CHUNK), jnp.int32),
        pltpu.VMEM((K, CHUNK), jnp.int32),
        pltpu.VMEM((CHUNK, D), jnp.float32),
        pltpu.VMEM((CHUNK,), jnp.float32),
        pltpu.VMEM((ROWS_PER_TILE,), jnp.float32),
        pltpu.VMEM_SHARED((NP, D), jnp.float32),
        pltpu.VMEM_SHARED((NP,), jnp.float32),
        pltpu.SemaphoreType.DMA,
    ]

    @functools.partial(pl.kernel, mesh=mesh, out_type=out_type,
                       scratch_types=scratch)
    def body(x_hbm, src_hbm, dst_hbm, sums_out, *rest):
        if with_deg:
            deg_out = rest[0]
            (src_v, dst_v, rows_v, ones_v, zdeg_v, acc_sh, deg_sh, sem) = rest[1:]
        else:
            (src_v, dst_v, rows_v, ones_v, zdeg_v, acc_sh, deg_sh, sem) = rest

        cid = lax.axis_index("c")
        sid = lax.axis_index("s")
        wid = cid * NUM_SUBCORES + sid

        zeros16 = jnp.zeros((16,), jnp.float32)

        def zero_row(i, _):
            for j in range(D // 16):
                rows_v[i, pl.ds(j * 16, 16)] = zeros16
            return 0

        lax.fori_loop(0, CHUNK, zero_row, 0)

        def zero_deg(i, _):
            zdeg_v[pl.ds(i * 16, 16)] = zeros16
            return 0

        lax.fori_loop(0, ROWS_PER_TILE // 16, zero_deg, 0)

        def fill_ones(i, _):
            ones_v[pl.ds(i * 16, 16)] = zeros16 + 1.0
            return 0

        lax.fori_loop(0, CHUNK // 16, fill_ones, 0)

        row0 = sid * ROWS_PER_TILE
        for t in range(ROWS_PER_TILE // CHUNK):
            pltpu.sync_copy(rows_v, acc_sh.at[pl.ds(row0 + t * CHUNK, CHUNK)])
        pltpu.sync_copy(zdeg_v, deg_sh.at[pl.ds(row0, ROWS_PER_TILE)])

        pltpu.sync_copy(src_hbm.at[pl.ds(wid * K, K)], src_v)
        pltpu.sync_copy(dst_hbm.at[pl.ds(wid * K, K)], dst_v)

        plsc.subcore_barrier()

        def step(j, _):
            pltpu.async_copy(x_hbm.at[src_v.at[j]], rows_v, sem).wait()
            pltpu.sync_copy(rows_v, acc_sh.at[dst_v.at[j]], add=True)
            if with_deg:
                pltpu.sync_copy(ones_v, deg_sh.at[dst_v.at[j]], add=True)
            return 0

        lax.fori_loop(0, K, step, 0)

        plsc.subcore_barrier()

        pltpu.sync_copy(acc_sh.at[pl.ds(row0, ROWS_PER_TILE)],
                        sums_out.at[cid, pl.ds(row0, ROWS_PER_TILE)])
        if with_deg:
            pltpu.sync_copy(deg_sh.at[pl.ds(row0, ROWS_PER_TILE)],
                            deg_out.at[cid, pl.ds(row0, ROWS_PER_TILE)])

    return body


_sc_sum_deg = _sc_segment_sum(True)
_sc_sum = _sc_segment_sum(False)

BN = 1280


def _tc_layer_body(do_relu, sums_ref, deg_ref, x_ref, wl_ref, b_ref, wr_ref,
                   out_ref):
    s = sums_ref[0] + sums_ref[1]
    d = deg_ref[0] + deg_ref[1]
    inv = 1.0 / jnp.maximum(d, 1.0)
    agg = s * inv[:, None]
    r = (jnp.dot(agg, wl_ref[...], preferred_element_type=jnp.float32)
         + b_ref[...]
         + jnp.dot(x_ref[...], wr_ref[...], preferred_element_type=jnp.float32))
    if do_relu:
        r = jnp.maximum(r, 0.0)
    out_ref[...] = r


def _tc_layer(sums, deg, x, wl_t, b, wr_t, do_relu):
    grid = (NP // BN,)
    return pl.pallas_call(
        functools.partial(_tc_layer_body, do_relu),
        grid=grid,
        in_specs=[
            pl.BlockSpec((NUM_CORES, BN, D), lambda i: (0, i, 0)),
            pl.BlockSpec((NUM_CORES, BN), lambda i: (0, i)),
            pl.BlockSpec((BN, D), lambda i: (i, 0)),
            pl.BlockSpec((D, D), lambda i: (0, 0)),
            pl.BlockSpec((1, D), lambda i: (0, 0)),
            pl.BlockSpec((D, D), lambda i: (0, 0)),
        ],
        out_specs=pl.BlockSpec((BN, D), lambda i: (i, 0)),
        out_shape=jax.ShapeDtypeStruct((NP, D), jnp.float32),
    )(sums, deg, x, wl_t, b, wr_t)


def kernel(x, edge_index, W_l1, b_l1, W_r1, W_l2, b_l2, W_r2):
    src = edge_index[0]
    dst = edge_index[1]

    pad_n = PE - E
    pad_ar = jnp.arange(pad_n, dtype=jnp.int32)
    src_p = jnp.concatenate([src, pad_ar % N]).reshape(NW * K, CHUNK)
    dst_p = jnp.concatenate([dst, N + pad_ar % (NP - N)]).reshape(NW * K, CHUNK)

    x_p = jnp.concatenate([x, jnp.zeros((NP - N, D), jnp.float32)], axis=0)

    sums1, deg = _sc_sum_deg(x_p, src_p, dst_p)
    h = _tc_layer(sums1, deg, x_p, W_l1.T, b_l1.reshape(1, D), W_r1.T, True)
    (sums2,) = _sc_sum(h, src_p, dst_p)
    out = _tc_layer(sums2, deg, h, W_l2.T, b_l2.reshape(1, D), W_r2.T, False)
    return out[:N]


# speedup vs baseline: 10.8802x; 1.1827x over previous
import functools

import jax
import jax.numpy as jnp
from jax import lax
from jax.experimental import pallas as pl
from jax.experimental.pallas import tpu as pltpu
from jax.experimental.pallas import tpu_sc as plsc

N = 10000
E = 320000
D = 128

NUM_CORES = 2
NUM_SUBCORES = 16
NW = NUM_CORES * NUM_SUBCORES
CHUNK = 128
K = 80
EW = K * CHUNK
PE = NW * EW
NP = 10240
NBUF = 2
PH = 2
KP = K // PH

ROWS_PER_TILE = NP // NUM_SUBCORES


def _sc_segment_sum(with_deg: bool):
    mesh = plsc.VectorSubcoreMesh(core_axis_name="c", subcore_axis_name="s")
    out_type = [jax.ShapeDtypeStruct((NUM_CORES, NP, D), jnp.float32)]
    if with_deg:
        out_type.append(jax.ShapeDtypeStruct((NUM_CORES, NP), jnp.float32))

    scratch = [
        pltpu.VMEM((KP, CHUNK), jnp.int32),
        pltpu.VMEM((KP, CHUNK), jnp.int32),
        pltpu.VMEM((NBUF, CHUNK, D), jnp.float32),
        pltpu.VMEM((CHUNK,), jnp.float32),
        pltpu.VMEM((ROWS_PER_TILE,), jnp.float32),
        pltpu.VMEM_SHARED((NP, D), jnp.float32),
        pltpu.VMEM_SHARED((NP,), jnp.float32),
    ] + [pltpu.SemaphoreType.DMA] * (2 * NBUF)

    @functools.partial(pl.kernel, mesh=mesh, out_type=out_type,
                       scratch_types=scratch)
    def body(x_hbm, src_hbm, dst_hbm, sums_out, *rest):
        if with_deg:
            deg_out = rest[0]
            rest = rest[1:]
        (src_v, dst_v, rows_v, ones_v, zdeg_v, acc_sh, deg_sh) = rest[:7]
        sem_g = rest[7:7 + NBUF]
        sem_s = rest[7 + NBUF:7 + 2 * NBUF]

        cid = lax.axis_index("c")
        sid = lax.axis_index("s")
        wid = cid * NUM_SUBCORES + sid

        zeros16 = jnp.zeros((16,), jnp.float32)

        def zero_row(i, _):
            for j in range(D // 16):
                rows_v[0, i, pl.ds(j * 16, 16)] = zeros16
            return 0

        lax.fori_loop(0, CHUNK, zero_row, 0)

        def zero_deg(i, _):
            zdeg_v[pl.ds(i * 16, 16)] = zeros16
            return 0

        lax.fori_loop(0, ROWS_PER_TILE // 16, zero_deg, 0)

        def fill_ones(i, _):
            ones_v[pl.ds(i * 16, 16)] = zeros16 + 1.0
            return 0

        lax.fori_loop(0, CHUNK // 16, fill_ones, 0)

        row0 = sid * ROWS_PER_TILE
        for t in range(ROWS_PER_TILE // CHUNK):
            pltpu.sync_copy(rows_v.at[0],
                            acc_sh.at[pl.ds(row0 + t * CHUNK, CHUNK)])
        pltpu.sync_copy(zdeg_v, deg_sh.at[pl.ds(row0, ROWS_PER_TILE)])

        plsc.subcore_barrier()

        for p in range(PH):
            pltpu.sync_copy(src_hbm.at[pl.ds(wid * K + p * KP, KP)], src_v)
            pltpu.sync_copy(dst_hbm.at[pl.ds(wid * K + p * KP, KP)], dst_v)
            for b in range(NBUF):
                pltpu.async_copy(x_hbm.at[src_v.at[b]], rows_v.at[b], sem_g[b])

            def step(jj, _):
                j = jj * NBUF
                for b in range(NBUF):
                    c = j + b
                    pltpu.make_async_copy(
                        x_hbm.at[src_v.at[c]], rows_v.at[b], sem_g[b]).wait()
                    pltpu.async_copy(
                        rows_v.at[b], acc_sh.at[dst_v.at[c]], sem_s[b],
                        add=True)
                    if with_deg:
                        pltpu.async_copy(
                            ones_v, deg_sh.at[dst_v.at[c]], sem_s[b], add=True)
                for b in range(NBUF):
                    c = j + b
                    n = c + NBUF
                    pltpu.make_async_copy(
                        rows_v.at[b], acc_sh.at[dst_v.at[c]], sem_s[b]).wait()
                    if with_deg:
                        pltpu.make_async_copy(
                            ones_v, deg_sh.at[dst_v.at[c]], sem_s[b]).wait()

                    @pl.when(n < KP)
                    def _():
                        pltpu.async_copy(
                            x_hbm.at[src_v.at[n]], rows_v.at[b], sem_g[b])
                return 0

            lax.fori_loop(0, KP // NBUF, step, 0)

        plsc.subcore_barrier()

        pltpu.sync_copy(acc_sh.at[pl.ds(row0, ROWS_PER_TILE)],
                        sums_out.at[cid, pl.ds(row0, ROWS_PER_TILE)])
        if with_deg:
            pltpu.sync_copy(deg_sh.at[pl.ds(row0, ROWS_PER_TILE)],
                            deg_out.at[cid, pl.ds(row0, ROWS_PER_TILE)])

    return body


_sc_sum_deg = _sc_segment_sum(True)
_sc_sum = _sc_segment_sum(False)

BN = 1280


def _tc_layer_body(do_relu, sums_ref, deg_ref, x_ref, wl_ref, b_ref, wr_ref,
                   out_ref):
    s = sums_ref[0] + sums_ref[1]
    d = deg_ref[0] + deg_ref[1]
    inv = 1.0 / jnp.maximum(d, 1.0)
    agg = s * inv[:, None]
    r = (jnp.dot(agg, wl_ref[...], preferred_element_type=jnp.float32)
         + b_ref[...]
         + jnp.dot(x_ref[...], wr_ref[...], preferred_element_type=jnp.float32))
    if do_relu:
        r = jnp.maximum(r, 0.0)
    out_ref[...] = r


def _tc_layer(sums, deg, x, wl_t, b, wr_t, do_relu):
    grid = (NP // BN,)
    return pl.pallas_call(
        functools.partial(_tc_layer_body, do_relu),
        grid=grid,
        in_specs=[
            pl.BlockSpec((NUM_CORES, BN, D), lambda i: (0, i, 0)),
            pl.BlockSpec((NUM_CORES, BN), lambda i: (0, i)),
            pl.BlockSpec((BN, D), lambda i: (i, 0)),
            pl.BlockSpec((D, D), lambda i: (0, 0)),
            pl.BlockSpec((1, D), lambda i: (0, 0)),
            pl.BlockSpec((D, D), lambda i: (0, 0)),
        ],
        out_specs=pl.BlockSpec((BN, D), lambda i: (i, 0)),
        out_shape=jax.ShapeDtypeStruct((NP, D), jnp.float32),
    )(sums, deg, x, wl_t, b, wr_t)


def kernel(x, edge_index, W_l1, b_l1, W_r1, W_l2, b_l2, W_r2):
    src = edge_index[0]
    dst = edge_index[1]

    pad_n = PE - E
    pad_ar = jnp.arange(pad_n, dtype=jnp.int32)
    src_p = jnp.concatenate([src, pad_ar % N]).reshape(NW * K, CHUNK)
    dst_p = jnp.concatenate([dst, N + pad_ar % (NP - N)]).reshape(NW * K, CHUNK)

    x_p = jnp.concatenate([x, jnp.zeros((NP - N, D), jnp.float32)], axis=0)

    sums1, deg = _sc_sum_deg(x_p, src_p, dst_p)
    h = _tc_layer(sums1, deg, x_p, W_l1.T, b_l1.reshape(1, D), W_r1.T, True)
    (sums2,) = _sc_sum(h, src_p, dst_p)
    out = _tc_layer(sums2, deg, h, W_l2.T, b_l2.reshape(1, D), W_r2.T, False)
    return out[:N]


# speedup vs baseline: 11.9402x; 1.0974x over previous
import functools

import jax
import jax.numpy as jnp
from jax import lax
from jax.experimental import pallas as pl
from jax.experimental.pallas import tpu as pltpu
from jax.experimental.pallas import tpu_sc as plsc

N = 10000
E = 320000
D = 128

NUM_CORES = 2
NUM_SUBCORES = 16
NW = NUM_CORES * NUM_SUBCORES
CHUNK = 128
K = 80
EW = K * CHUNK
PE = NW * EW
NP = 10240
NBUF = 2
DEPTH = NBUF // 2
PH = 2
KP = K // PH

ROWS_PER_TILE = NP // NUM_SUBCORES


def _sc_segment_sum(with_deg: bool):
    mesh = plsc.VectorSubcoreMesh(core_axis_name="c", subcore_axis_name="s")
    out_type = [jax.ShapeDtypeStruct((NUM_CORES, NP, D), jnp.float32)]
    if with_deg:
        out_type.append(jax.ShapeDtypeStruct((NUM_CORES, NP), jnp.float32))

    scratch = [
        pltpu.VMEM((KP, CHUNK), jnp.int32),
        pltpu.VMEM((KP, CHUNK), jnp.int32),
        pltpu.VMEM((NBUF, CHUNK, D), jnp.float32),
        pltpu.VMEM((CHUNK,), jnp.float32),
        pltpu.VMEM((ROWS_PER_TILE,), jnp.float32),
        pltpu.VMEM_SHARED((NP, D), jnp.float32),
        pltpu.VMEM_SHARED((NP,), jnp.float32),
    ] + [pltpu.SemaphoreType.DMA] * (2 * NBUF)

    @functools.partial(pl.kernel, mesh=mesh, out_type=out_type,
                       scratch_types=scratch)
    def body(x_hbm, src_hbm, dst_hbm, sums_out, *rest):
        if with_deg:
            deg_out = rest[0]
            rest = rest[1:]
        (src_v, dst_v, rows_v, ones_v, zdeg_v, acc_sh, deg_sh) = rest[:7]
        sem_g = rest[7:7 + NBUF]
        sem_s = rest[7 + NBUF:7 + 2 * NBUF]

        cid = lax.axis_index("c")
        sid = lax.axis_index("s")
        wid = cid * NUM_SUBCORES + sid

        zeros16 = jnp.zeros((16,), jnp.float32)

        def zero_row(i, _):
            for j in range(D // 16):
                rows_v[0, i, pl.ds(j * 16, 16)] = zeros16
            return 0

        lax.fori_loop(0, CHUNK, zero_row, 0)

        def zero_deg(i, _):
            zdeg_v[pl.ds(i * 16, 16)] = zeros16
            return 0

        lax.fori_loop(0, ROWS_PER_TILE // 16, zero_deg, 0)

        def fill_ones(i, _):
            ones_v[pl.ds(i * 16, 16)] = zeros16 + 1.0
            return 0

        lax.fori_loop(0, CHUNK // 16, fill_ones, 0)

        row0 = sid * ROWS_PER_TILE
        for t in range(ROWS_PER_TILE // CHUNK):
            pltpu.sync_copy(rows_v.at[0],
                            acc_sh.at[pl.ds(row0 + t * CHUNK, CHUNK)])
        pltpu.sync_copy(zdeg_v, deg_sh.at[pl.ds(row0, ROWS_PER_TILE)])

        plsc.subcore_barrier()

        for p in range(PH):
            pltpu.sync_copy(src_hbm.at[pl.ds(wid * K + p * KP, KP)], src_v)
            pltpu.sync_copy(dst_hbm.at[pl.ds(wid * K + p * KP, KP)], dst_v)
            for b in range(DEPTH):
                pltpu.async_copy(x_hbm.at[src_v.at[b]], rows_v.at[b], sem_g[b])

            def step(jj, _):
                j = jj * NBUF
                for u in range(NBUF):
                    c = j + u
                    bn = (u + DEPTH) % NBUF
                    pltpu.make_async_copy(
                        x_hbm.at[src_v.at[c]], rows_v.at[u], sem_g[u]).wait()

                    @pl.when(c >= DEPTH)
                    def _():
                        pltpu.make_async_copy(
                            rows_v.at[bn], acc_sh.at[dst_v.at[c - DEPTH]],
                            sem_s[bn]).wait()
                        if with_deg:
                            pltpu.make_async_copy(
                                ones_v, deg_sh.at[dst_v.at[c - DEPTH]],
                                sem_s[bn]).wait()

                    pltpu.async_copy(
                        rows_v.at[u], acc_sh.at[dst_v.at[c]], sem_s[u],
                        add=True)
                    if with_deg:
                        pltpu.async_copy(
                            ones_v, deg_sh.at[dst_v.at[c]], sem_s[u], add=True)

                    @pl.when(c + DEPTH < KP)
                    def _():
                        pltpu.async_copy(
                            x_hbm.at[src_v.at[c + DEPTH]], rows_v.at[bn],
                            sem_g[bn])
                return 0

            lax.fori_loop(0, KP // NBUF, step, 0)

            for u in range(DEPTH):
                c = KP - DEPTH + u
                b = c % NBUF
                pltpu.make_async_copy(
                    rows_v.at[b], acc_sh.at[dst_v.at[c]], sem_s[b]).wait()
                if with_deg:
                    pltpu.make_async_copy(
                        ones_v, deg_sh.at[dst_v.at[c]], sem_s[b]).wait()

        plsc.subcore_barrier()

        pltpu.sync_copy(acc_sh.at[pl.ds(row0, ROWS_PER_TILE)],
                        sums_out.at[cid, pl.ds(row0, ROWS_PER_TILE)])
        if with_deg:
            pltpu.sync_copy(deg_sh.at[pl.ds(row0, ROWS_PER_TILE)],
                            deg_out.at[cid, pl.ds(row0, ROWS_PER_TILE)])

    return body


_sc_sum_deg = _sc_segment_sum(True)
_sc_sum = _sc_segment_sum(False)

BN = 1280


def _tc_layer_body(do_relu, sums_ref, deg_ref, x_ref, wl_ref, b_ref, wr_ref,
                   out_ref):
    s = sums_ref[0] + sums_ref[1]
    d = deg_ref[0] + deg_ref[1]
    inv = 1.0 / jnp.maximum(d, 1.0)
    agg = s * inv[:, None]
    r = (jnp.dot(agg, wl_ref[...], preferred_element_type=jnp.float32)
         + b_ref[...]
         + jnp.dot(x_ref[...], wr_ref[...], preferred_element_type=jnp.float32))
    if do_relu:
        r = jnp.maximum(r, 0.0)
    out_ref[...] = r


def _tc_layer(sums, deg, x, wl_t, b, wr_t, do_relu):
    grid = (NP // BN,)
    return pl.pallas_call(
        functools.partial(_tc_layer_body, do_relu),
        grid=grid,
        in_specs=[
            pl.BlockSpec((NUM_CORES, BN, D), lambda i: (0, i, 0)),
            pl.BlockSpec((NUM_CORES, BN), lambda i: (0, i)),
            pl.BlockSpec((BN, D), lambda i: (i, 0)),
            pl.BlockSpec((D, D), lambda i: (0, 0)),
            pl.BlockSpec((1, D), lambda i: (0, 0)),
            pl.BlockSpec((D, D), lambda i: (0, 0)),
        ],
        out_specs=pl.BlockSpec((BN, D), lambda i: (i, 0)),
        out_shape=jax.ShapeDtypeStruct((NP, D), jnp.float32),
    )(sums, deg, x, wl_t, b, wr_t)


def kernel(x, edge_index, W_l1, b_l1, W_r1, W_l2, b_l2, W_r2):
    src = edge_index[0]
    dst = edge_index[1]

    pad_n = PE - E
    pad_ar = jnp.arange(pad_n, dtype=jnp.int32)
    src_p = jnp.concatenate([src, pad_ar % N]).reshape(NW * K, CHUNK)
    dst_p = jnp.concatenate([dst, N + pad_ar % (NP - N)]).reshape(NW * K, CHUNK)

    x_p = jnp.concatenate([x, jnp.zeros((NP - N, D), jnp.float32)], axis=0)

    sums1, deg = _sc_sum_deg(x_p, src_p, dst_p)
    h = _tc_layer(sums1, deg, x_p, W_l1.T, b_l1.reshape(1, D), W_r1.T, True)
    (sums2,) = _sc_sum(h, src_p, dst_p)
    out = _tc_layer(sums2, deg, h, W_l2.T, b_l2.reshape(1, D), W_r2.T, False)
    return out[:N]
